# Initial kernel scaffold; baseline (speedup 1.0000x reference)
import jax
import jax.numpy as jnp
from jax.experimental import pallas as pl


def kernel(y_pred, y_true, lut):
    raise NotImplementedError("write your pallas kernel here")


# speedup vs baseline: 745.9897x; 745.9897x over previous
import jax
import jax.numpy as jnp
from jax.experimental import pallas as pl

_SDF_MIN = -7.0
_SDF_MAX = 7.0
_N_BINS = 256


def _body(yp_ref, yt_ref, lut_ref, out_ref):
    i = pl.program_id(0)
    yp = yp_ref[...]
    yt = yt_ref[...]
    t = jnp.clip(yt, _SDF_MIN, _SDF_MAX)
    unit = (t - _SDF_MIN) * (1.0 / (_SDF_MAX - _SDF_MIN))
    x = unit * (_N_BINS - 1)
    idx = (x + 0.5).astype(jnp.int32)
    br = idx.shape[0]
    lo = jnp.broadcast_to(lut_ref[0:128][None, :], (br, 128))
    hi = jnp.broadcast_to(lut_ref[128:256][None, :], (br, 128))
    idxm = idx & 127
    wlo = jnp.take_along_axis(lo, idxm, axis=1)
    whi = jnp.take_along_axis(hi, idxm, axis=1)
    w = jnp.where(idx < 128, wlo, whi)
    d = yp - yt
    s = jnp.sum(w * (d * d))

    @pl.when(i == 0)
    def _():
        out_ref[...] = jnp.zeros((1, 1), jnp.float32)

    out_ref[...] += jnp.full((1, 1), 1.0, jnp.float32) * s


def kernel(y_pred, y_true, lut):
    n = y_pred.size
    rows, cols = 8192, 2048
    yp = y_pred.reshape(rows, cols)
    yt = y_true.reshape(rows, cols)
    grid = 16
    br = rows // grid
    total = pl.pallas_call(
        _body,
        grid=(grid,),
        in_specs=[
            pl.BlockSpec((br, cols), lambda i: (i, 0)),
            pl.BlockSpec((br, cols), lambda i: (i, 0)),
            pl.BlockSpec((_N_BINS,), lambda i: (0,)),
        ],
        out_specs=pl.BlockSpec((1, 1), lambda i: (0, 0)),
        out_shape=jax.ShapeDtypeStruct((1, 1), jnp.float32),
    )(yp, yt, lut)
    return (total[0, 0] / n).astype(jnp.float32)
